# baseline (speedup 1.0000x reference)
import jax
import jax.numpy as jnp
from jax import lax
from jax.experimental import pallas as pl
from jax.experimental.pallas import tpu as pltpu
from jax.experimental.pallas import tpu_sc as plsc

USER = 25000
ITEM = 25000
N = USER + ITEM
D = 64
H = 128
HALF = 32
LEAKY = 0.5
E = 800000

EPT = 50176
EPAD = 16 * EPT
PADN = EPAD - E
WIN = 256
NSUB = 2
NWIN = EPT // WIN
NP = 50048
RPT = NP // 16
ZR = 68

BLK = 1000
GRID = N // BLK


def _lrelu(x):
    return jnp.where(x >= 0, x, LEAKY * x)


def _spmm_body(x_hbm, cols_hbm, rows_hbm, vals_hbm, out_hbm,
               gbuf0, gbuf1, cbuf0, cbuf1, rbuf0, rbuf1, rsbuf0, rsbuf1,
               vbuf0, vbuf1, zbuf, acc, isem0, isem1, gsem0, gsem1,
               ssem0, ssem1):
    c = lax.axis_index("c")
    s = lax.axis_index("s")

    def _zrow(r, _):
        zbuf[r, pl.ds(0, 16)] = jnp.zeros((16,), jnp.float32)
        zbuf[r, pl.ds(16, 16)] = jnp.zeros((16,), jnp.float32)
        return _
    lax.fori_loop(0, ZR, _zrow, None)

    def _zcopy(j, _):
        pltpu.sync_copy(zbuf, acc.at[pl.ds(s * RPT + j * ZR, ZR)])
        return _
    lax.fori_loop(0, RPT // ZR, _zcopy, None)
    plsc.subcore_barrier()

    col_off = c * N

    def _idx_slices(w):
        base = s * EPT + w * WIN
        chunk = s * (EPT // 128) + w * NSUB
        return (vals_hbm.at[pl.ds(base, WIN)],
                cols_hbm.at[pl.ds(chunk, NSUB)],
                rows_hbm.at[pl.ds(chunk, NSUB)])

    def _issue_idx(w, cb, rb, vb, sem):
        vsrc, csrc, rsrc = _idx_slices(w)
        pltpu.async_copy(vsrc, vb, sem)
        pltpu.async_copy(csrc, cb, sem)
        pltpu.async_copy(rsrc, rb, sem)

    def _drain_idx(w, cb, rb, vb, sem):
        vsrc, csrc, rsrc = _idx_slices(w)
        pltpu.make_async_copy(vsrc, vb, sem).wait()
        pltpu.make_async_copy(csrc, cb, sem).wait()
        pltpu.make_async_copy(rsrc, rb, sem).wait()

    def _adjust(cb):
        for t in range(NSUB):
            for l in range(8):
                sl = pl.ds(l * 16, 16)
                cb[t, sl] = cb[t, sl] + col_off

    def _gathers(cb, gb, sem):
        return [
            pltpu.async_copy(x_hbm.at[cb.at[t]],
                             gb.at[pl.ds(t * 128, 128)], sem)
            for t in range(NSUB)
        ]

    def _copy_rows(rb, rsb):
        for t in range(NSUB):
            for l in range(8):
                sl = pl.ds(l * 16, 16)
                rsb[t, sl] = rb[t, sl]

    def _scale_scatter_async(gb, vb, rsb, ssem):
        for t in range(NSUB):
            @plsc.parallel_loop(t * 8, (t + 1) * 8, unroll=4)
            def _scale(g):
                v16 = vb[pl.ds(g * 16, 16)]
                for r in range(16):
                    e = g * 16 + r
                    lo = pl.ds(0, 16)
                    hi = pl.ds(16, 16)
                    gb[e, lo] = gb[e, lo] * v16[r]
                    gb[e, hi] = gb[e, hi] * v16[r]

            pltpu.async_copy(gb.at[pl.ds(t * 128, 128)],
                             acc.at[rsb.at[t]], ssem, add=True)

    def _drain_scatter(gb, rsb, ssem):
        for t in range(NSUB):
            pltpu.make_async_copy(gb.at[pl.ds(t * 128, 128)],
                                  acc.at[rsb.at[t]], ssem).wait()

    _issue_idx(0, cbuf0, rbuf0, vbuf0, isem0)

    def _pair(k, _):
        w0 = 2 * k
        w1 = w0 + 1

        @pl.when(k > 0)
        def _():
            _drain_scatter(gbuf0, rsbuf0, ssem0)
        _drain_idx(w0, cbuf0, rbuf0, vbuf0, isem0)
        _adjust(cbuf0)
        g0 = _gathers(cbuf0, gbuf0, gsem0)
        _issue_idx(w1, cbuf1, rbuf1, vbuf1, isem1)
        for d in g0:
            d.wait()
        _copy_rows(rbuf0, rsbuf0)

        @pl.when(k > 0)
        def _():
            _drain_scatter(gbuf1, rsbuf1, ssem1)
        _drain_idx(w1, cbuf1, rbuf1, vbuf1, isem1)
        _adjust(cbuf1)
        g1 = _gathers(cbuf1, gbuf1, gsem1)

        _scale_scatter_async(gbuf0, vbuf0, rsbuf0, ssem0)

        @pl.when(k < NWIN // 2 - 1)
        def _():
            _issue_idx(w0 + 2, cbuf0, rbuf0, vbuf0, isem0)

        for d in g1:
            d.wait()
        _copy_rows(rbuf1, rsbuf1)
        _scale_scatter_async(gbuf1, vbuf1, rsbuf1, ssem1)
        return _
    lax.fori_loop(0, NWIN // 2, _pair, None)
    _drain_scatter(gbuf0, rsbuf0, ssem0)
    _drain_scatter(gbuf1, rsbuf1, ssem1)
    plsc.subcore_barrier()

    pltpu.sync_copy(acc.at[pl.ds(s * RPT, RPT)],
                    out_hbm.at[pl.ds(c * NP + s * RPT, RPT)])


_spmm_sc = pl.kernel(
    _spmm_body,
    out_type=jax.ShapeDtypeStruct((2 * NP, HALF), jnp.float32),
    mesh=plsc.VectorSubcoreMesh(core_axis_name="c", subcore_axis_name="s"),
    scratch_types=[
        pltpu.VMEM((WIN, HALF), jnp.float32),
        pltpu.VMEM((WIN, HALF), jnp.float32),
        pltpu.VMEM((NSUB, 128), jnp.int32),
        pltpu.VMEM((NSUB, 128), jnp.int32),
        pltpu.VMEM((NSUB, 128), jnp.int32),
        pltpu.VMEM((NSUB, 128), jnp.int32),
        pltpu.VMEM((NSUB, 128), jnp.int32),
        pltpu.VMEM((NSUB, 128), jnp.int32),
        pltpu.VMEM((WIN,), jnp.float32),
        pltpu.VMEM((WIN,), jnp.float32),
        pltpu.VMEM((ZR, HALF), jnp.float32),
        pltpu.VMEM_SHARED((NP, HALF), jnp.float32),
        pltpu.SemaphoreType.DMA,
        pltpu.SemaphoreType.DMA,
        pltpu.SemaphoreType.DMA,
        pltpu.SemaphoreType.DMA,
        pltpu.SemaphoreType.DMA,
        pltpu.SemaphoreType.DMA,
    ],
    compiler_params=pltpu.CompilerParams(use_tc_tiling_on_sc=False),
)


def _stage_a_body(e_ref, z_ref, h_ref, ah_ref, x_ref, lm_ref):
    i = pl.program_id(0)
    e = e_ref[...]
    a = jax.nn.sigmoid(z_ref[...]) * 2.0 - 1.0
    x = e * a
    ah = jnp.dot(e, h_ref[...], preferred_element_type=jnp.float32)
    ah_ref[...] = ah
    x_ref[0] = x[:, :HALF]
    x_ref[1] = x[:, HALF:]

    @pl.when(i == 0)
    def _():
        lm_ref[...] = jnp.zeros_like(lm_ref)

    lm_ref[...] += lax.dot_general(
        ah, e, (((0,), (0,)), ((), ())), preferred_element_type=jnp.float32)

    @pl.when(i == GRID - 1)
    def _():
        lm_ref[...] = _lrelu(lm_ref[...])


_stage_a = pl.pallas_call(
    _stage_a_body,
    grid=(GRID,),
    in_specs=[
        pl.BlockSpec((BLK, D), lambda i: (i, 0)),
        pl.BlockSpec((BLK, D), lambda i: (i, 0)),
        pl.BlockSpec((D, H), lambda i: (0, 0)),
    ],
    out_specs=[
        pl.BlockSpec((BLK, H), lambda i: (i, 0)),
        pl.BlockSpec((2, BLK, HALF), lambda i: (0, i, 0)),
        pl.BlockSpec((H, D), lambda i: (0, 0)),
    ],
    out_shape=[
        jax.ShapeDtypeStruct((N, H), jnp.float32),
        jax.ShapeDtypeStruct((2, N, HALF), jnp.float32),
        jax.ShapeDtypeStruct((H, D), jnp.float32),
    ],
)


def _stage_h_body(ah_ref, lm_ref, hl_ref):
    hl_ref[...] = _lrelu(jnp.dot(ah_ref[...], lm_ref[...],
                                 preferred_element_type=jnp.float32))


_stage_h = pl.pallas_call(
    _stage_h_body,
    grid=(GRID,),
    in_specs=[
        pl.BlockSpec((BLK, H), lambda i: (i, 0)),
        pl.BlockSpec((H, D), lambda i: (0, 0)),
    ],
    out_specs=pl.BlockSpec((BLK, D), lambda i: (i, 0)),
    out_shape=jax.ShapeDtypeStruct((N, D), jnp.float32),
)


def _stage_cb_body(hl_ref, s_ref, z_ref, ah_ref, x_ref, lm_ref):
    i = pl.program_id(0)
    t = _lrelu(jnp.concatenate([s_ref[0], s_ref[1]], axis=1))
    lat = hl_ref[...] + t
    a = jax.nn.sigmoid(z_ref[...]) * 2.0 - 1.0
    x = lat * a
    x_ref[0] = x[:, :HALF]
    x_ref[1] = x[:, HALF:]

    @pl.when(i == 0)
    def _():
        lm_ref[...] = jnp.zeros_like(lm_ref)

    lm_ref[...] += lax.dot_general(
        ah_ref[...], lat, (((0,), (0,)), ((), ())),
        preferred_element_type=jnp.float32)

    @pl.when(i == GRID - 1)
    def _():
        lm_ref[...] = _lrelu(lm_ref[...])


_stage_cb = pl.pallas_call(
    _stage_cb_body,
    grid=(GRID,),
    in_specs=[
        pl.BlockSpec((BLK, D), lambda i: (i, 0)),
        pl.BlockSpec((2, BLK, HALF), lambda i: (0, i, 0)),
        pl.BlockSpec((BLK, D), lambda i: (i, 0)),
        pl.BlockSpec((BLK, H), lambda i: (i, 0)),
    ],
    out_specs=[
        pl.BlockSpec((2, BLK, HALF), lambda i: (0, i, 0)),
        pl.BlockSpec((H, D), lambda i: (0, 0)),
    ],
    out_shape=[
        jax.ShapeDtypeStruct((2, N, HALF), jnp.float32),
        jax.ShapeDtypeStruct((H, D), jnp.float32),
    ],
)


def _stage_eb_body(s_ref, te_ref):
    te_ref[...] = _lrelu(jnp.concatenate([s_ref[0], s_ref[1]], axis=1))


_stage_eb = pl.pallas_call(
    _stage_eb_body,
    grid=(GRID,),
    in_specs=[
        pl.BlockSpec((2, BLK, HALF), lambda i: (0, i, 0)),
    ],
    out_specs=pl.BlockSpec((BLK, D), lambda i: (i, 0)),
    out_shape=jax.ShapeDtypeStruct((N, D), jnp.float32),
)


def kernel(adj_indices, adj_values, keepRate, uEmbeds, iEmbeds, Hyper,
           zishiying):
    del keepRate
    embeds = jnp.concatenate([uEmbeds, iEmbeds], axis=0)

    pad_idx = (jnp.arange(PADN, dtype=jnp.int32) * 64) % N
    rows = jnp.concatenate([adj_indices[0], pad_idx]).reshape(EPAD // 128, 128)
    cols = jnp.concatenate([adj_indices[1], pad_idx]).reshape(EPAD // 128, 128)
    vals = jnp.concatenate(
        [adj_values, jnp.zeros((PADN,), jnp.float32)])

    allHyper, x1, lm1 = _stage_a(embeds, zishiying, Hyper)

    s1 = _spmm_sc(x1.reshape(2 * N, HALF), cols, rows, vals)
    hyperLat1 = _stage_h(allHyper, lm1)
    x2, lm2 = _stage_cb(hyperLat1, s1.reshape(2, NP, HALF), zishiying,
                        allHyper)

    s2 = _spmm_sc(x2.reshape(2 * N, HALF), cols, rows, vals)
    hyperLat2 = _stage_h(allHyper, lm2)
    temEmbeds2 = _stage_eb(s2.reshape(2, NP, HALF))

    return (temEmbeds2, hyperLat1, hyperLat2)


# speedup vs baseline: 1.0007x; 1.0007x over previous
import jax
import jax.numpy as jnp
from jax import lax
from jax.experimental import pallas as pl
from jax.experimental.pallas import tpu as pltpu
from jax.experimental.pallas import tpu_sc as plsc

USER = 25000
ITEM = 25000
N = USER + ITEM
D = 64
H = 128
HALF = 32
LEAKY = 0.5
E = 800000

EPT = 50176
EPAD = 16 * EPT
PADN = EPAD - E
WIN = 256
NSUB = 2
NWIN = EPT // WIN
NP = 50048
RPT = NP // 16
ZR = 68

BLK = 1000
GRID = N // BLK


def _lrelu(x):
    return jnp.where(x >= 0, x, LEAKY * x)


def _spmm_body(x_hbm, cols_hbm, rows_hbm, vals_hbm, out_hbm,
               gbuf0, gbuf1, cbuf0, cbuf1, rbuf0, rbuf1, rsbuf0, rsbuf1,
               vbuf0, vbuf1, zbuf, acc, isem0, isem1, gsem0, gsem1,
               ssem0, ssem1):
    c = lax.axis_index("c")
    s = lax.axis_index("s")

    def _zrow(r, _):
        zbuf[r, pl.ds(0, 16)] = jnp.zeros((16,), jnp.float32)
        zbuf[r, pl.ds(16, 16)] = jnp.zeros((16,), jnp.float32)
        return _
    lax.fori_loop(0, ZR, _zrow, None)

    def _zcopy(j, _):
        pltpu.sync_copy(zbuf, acc.at[pl.ds(s * RPT + j * ZR, ZR)])
        return _
    lax.fori_loop(0, RPT // ZR, _zcopy, None)
    plsc.subcore_barrier()

    col_off = c * N

    def _idx_slices(w):
        base = s * EPT + w * WIN
        chunk = s * (EPT // 128) + w * NSUB
        return (vals_hbm.at[pl.ds(base, WIN)],
                cols_hbm.at[pl.ds(chunk, NSUB)],
                rows_hbm.at[pl.ds(chunk, NSUB)])

    def _issue_idx(w, cb, rb, vb, sem):
        vsrc, csrc, rsrc = _idx_slices(w)
        pltpu.async_copy(vsrc, vb, sem)
        pltpu.async_copy(csrc, cb, sem)
        pltpu.async_copy(rsrc, rb, sem)

    def _drain_idx(w, cb, rb, vb, sem):
        vsrc, csrc, rsrc = _idx_slices(w)
        pltpu.make_async_copy(vsrc, vb, sem).wait()
        pltpu.make_async_copy(csrc, cb, sem).wait()
        pltpu.make_async_copy(rsrc, rb, sem).wait()

    def _adjust(cb):
        for t in range(NSUB):
            for l in range(8):
                sl = pl.ds(l * 16, 16)
                cb[t, sl] = cb[t, sl] + col_off

    def _gathers(cb, gb, sem):
        return [
            pltpu.async_copy(x_hbm.at[cb.at[t]],
                             gb.at[pl.ds(t * 128, 128)], sem)
            for t in range(NSUB)
        ]

    def _copy_rows(rb, rsb):
        for t in range(NSUB):
            for l in range(8):
                sl = pl.ds(l * 16, 16)
                rsb[t, sl] = rb[t, sl]

    def _scale_scatter_async(gb, vb, rsb, ssem):
        for t in range(NSUB):
            @plsc.parallel_loop(t * 8, (t + 1) * 8, unroll=4)
            def _scale(g):
                v16 = vb[pl.ds(g * 16, 16)]
                for r in range(16):
                    e = g * 16 + r
                    lo = pl.ds(0, 16)
                    hi = pl.ds(16, 16)
                    vs = lax.gather(
                        v16, jnp.full((16, 1), r, jnp.int32),
                        dimension_numbers=lax.GatherDimensionNumbers(
                            offset_dims=(), collapsed_slice_dims=(0,),
                            start_index_map=(0,)),
                        slice_sizes=(1,),
                        mode=lax.GatherScatterMode.PROMISE_IN_BOUNDS)
                    gb[e, lo] = gb[e, lo] * vs
                    gb[e, hi] = gb[e, hi] * vs

            pltpu.async_copy(gb.at[pl.ds(t * 128, 128)],
                             acc.at[rsb.at[t]], ssem, add=True)

    def _drain_scatter(gb, rsb, ssem):
        for t in range(NSUB):
            pltpu.make_async_copy(gb.at[pl.ds(t * 128, 128)],
                                  acc.at[rsb.at[t]], ssem).wait()

    _issue_idx(0, cbuf0, rbuf0, vbuf0, isem0)

    def _pair(k, _):
        w0 = 2 * k
        w1 = w0 + 1

        @pl.when(k > 0)
        def _():
            _drain_scatter(gbuf0, rsbuf0, ssem0)
        _drain_idx(w0, cbuf0, rbuf0, vbuf0, isem0)
        _adjust(cbuf0)
        g0 = _gathers(cbuf0, gbuf0, gsem0)
        _issue_idx(w1, cbuf1, rbuf1, vbuf1, isem1)
        for d in g0:
            d.wait()
        _copy_rows(rbuf0, rsbuf0)

        @pl.when(k > 0)
        def _():
            _drain_scatter(gbuf1, rsbuf1, ssem1)
        _drain_idx(w1, cbuf1, rbuf1, vbuf1, isem1)
        _adjust(cbuf1)
        g1 = _gathers(cbuf1, gbuf1, gsem1)

        _scale_scatter_async(gbuf0, vbuf0, rsbuf0, ssem0)

        @pl.when(k < NWIN // 2 - 1)
        def _():
            _issue_idx(w0 + 2, cbuf0, rbuf0, vbuf0, isem0)

        for d in g1:
            d.wait()
        _copy_rows(rbuf1, rsbuf1)
        _scale_scatter_async(gbuf1, vbuf1, rsbuf1, ssem1)
        return _
    lax.fori_loop(0, NWIN // 2, _pair, None)
    _drain_scatter(gbuf0, rsbuf0, ssem0)
    _drain_scatter(gbuf1, rsbuf1, ssem1)
    plsc.subcore_barrier()

    pltpu.sync_copy(acc.at[pl.ds(s * RPT, RPT)],
                    out_hbm.at[pl.ds(c * NP + s * RPT, RPT)])


_spmm_sc = pl.kernel(
    _spmm_body,
    out_type=jax.ShapeDtypeStruct((2 * NP, HALF), jnp.float32),
    mesh=plsc.VectorSubcoreMesh(core_axis_name="c", subcore_axis_name="s"),
    scratch_types=[
        pltpu.VMEM((WIN, HALF), jnp.float32),
        pltpu.VMEM((WIN, HALF), jnp.float32),
        pltpu.VMEM((NSUB, 128), jnp.int32),
        pltpu.VMEM((NSUB, 128), jnp.int32),
        pltpu.VMEM((NSUB, 128), jnp.int32),
        pltpu.VMEM((NSUB, 128), jnp.int32),
        pltpu.VMEM((NSUB, 128), jnp.int32),
        pltpu.VMEM((NSUB, 128), jnp.int32),
        pltpu.VMEM((WIN,), jnp.float32),
        pltpu.VMEM((WIN,), jnp.float32),
        pltpu.VMEM((ZR, HALF), jnp.float32),
        pltpu.VMEM_SHARED((NP, HALF), jnp.float32),
        pltpu.SemaphoreType.DMA,
        pltpu.SemaphoreType.DMA,
        pltpu.SemaphoreType.DMA,
        pltpu.SemaphoreType.DMA,
        pltpu.SemaphoreType.DMA,
        pltpu.SemaphoreType.DMA,
    ],
    compiler_params=pltpu.CompilerParams(use_tc_tiling_on_sc=False),
)


def _stage_a_body(e_ref, z_ref, h_ref, ah_ref, x_ref, lm_ref):
    i = pl.program_id(0)
    e = e_ref[...]
    a = jax.nn.sigmoid(z_ref[...]) * 2.0 - 1.0
    x = e * a
    ah = jnp.dot(e, h_ref[...], preferred_element_type=jnp.float32)
    ah_ref[...] = ah
    x_ref[0] = x[:, :HALF]
    x_ref[1] = x[:, HALF:]

    @pl.when(i == 0)
    def _():
        lm_ref[...] = jnp.zeros_like(lm_ref)

    lm_ref[...] += lax.dot_general(
        ah, e, (((0,), (0,)), ((), ())), preferred_element_type=jnp.float32)

    @pl.when(i == GRID - 1)
    def _():
        lm_ref[...] = _lrelu(lm_ref[...])


_stage_a = pl.pallas_call(
    _stage_a_body,
    grid=(GRID,),
    in_specs=[
        pl.BlockSpec((BLK, D), lambda i: (i, 0)),
        pl.BlockSpec((BLK, D), lambda i: (i, 0)),
        pl.BlockSpec((D, H), lambda i: (0, 0)),
    ],
    out_specs=[
        pl.BlockSpec((BLK, H), lambda i: (i, 0)),
        pl.BlockSpec((2, BLK, HALF), lambda i: (0, i, 0)),
        pl.BlockSpec((H, D), lambda i: (0, 0)),
    ],
    out_shape=[
        jax.ShapeDtypeStruct((N, H), jnp.float32),
        jax.ShapeDtypeStruct((2, N, HALF), jnp.float32),
        jax.ShapeDtypeStruct((H, D), jnp.float32),
    ],
)


def _stage_h_body(ah_ref, lm_ref, hl_ref):
    hl_ref[...] = _lrelu(jnp.dot(ah_ref[...], lm_ref[...],
                                 preferred_element_type=jnp.float32))


_stage_h = pl.pallas_call(
    _stage_h_body,
    grid=(GRID,),
    in_specs=[
        pl.BlockSpec((BLK, H), lambda i: (i, 0)),
        pl.BlockSpec((H, D), lambda i: (0, 0)),
    ],
    out_specs=pl.BlockSpec((BLK, D), lambda i: (i, 0)),
    out_shape=jax.ShapeDtypeStruct((N, D), jnp.float32),
)


def _stage_cb_body(hl_ref, s_ref, z_ref, ah_ref, x_ref, lm_ref):
    i = pl.program_id(0)
    t = _lrelu(jnp.concatenate([s_ref[0], s_ref[1]], axis=1))
    lat = hl_ref[...] + t
    a = jax.nn.sigmoid(z_ref[...]) * 2.0 - 1.0
    x = lat * a
    x_ref[0] = x[:, :HALF]
    x_ref[1] = x[:, HALF:]

    @pl.when(i == 0)
    def _():
        lm_ref[...] = jnp.zeros_like(lm_ref)

    lm_ref[...] += lax.dot_general(
        ah_ref[...], lat, (((0,), (0,)), ((), ())),
        preferred_element_type=jnp.float32)

    @pl.when(i == GRID - 1)
    def _():
        lm_ref[...] = _lrelu(lm_ref[...])


_stage_cb = pl.pallas_call(
    _stage_cb_body,
    grid=(GRID,),
    in_specs=[
        pl.BlockSpec((BLK, D), lambda i: (i, 0)),
        pl.BlockSpec((2, BLK, HALF), lambda i: (0, i, 0)),
        pl.BlockSpec((BLK, D), lambda i: (i, 0)),
        pl.BlockSpec((BLK, H), lambda i: (i, 0)),
    ],
    out_specs=[
        pl.BlockSpec((2, BLK, HALF), lambda i: (0, i, 0)),
        pl.BlockSpec((H, D), lambda i: (0, 0)),
    ],
    out_shape=[
        jax.ShapeDtypeStruct((2, N, HALF), jnp.float32),
        jax.ShapeDtypeStruct((H, D), jnp.float32),
    ],
)


def _stage_eb_body(s_ref, te_ref):
    te_ref[...] = _lrelu(jnp.concatenate([s_ref[0], s_ref[1]], axis=1))


_stage_eb = pl.pallas_call(
    _stage_eb_body,
    grid=(GRID,),
    in_specs=[
        pl.BlockSpec((2, BLK, HALF), lambda i: (0, i, 0)),
    ],
    out_specs=pl.BlockSpec((BLK, D), lambda i: (i, 0)),
    out_shape=jax.ShapeDtypeStruct((N, D), jnp.float32),
)


def kernel(adj_indices, adj_values, keepRate, uEmbeds, iEmbeds, Hyper,
           zishiying):
    del keepRate
    embeds = jnp.concatenate([uEmbeds, iEmbeds], axis=0)

    pad_idx = (jnp.arange(PADN, dtype=jnp.int32) * 64) % N
    rows = jnp.concatenate([adj_indices[0], pad_idx]).reshape(EPAD // 128, 128)
    cols = jnp.concatenate([adj_indices[1], pad_idx]).reshape(EPAD // 128, 128)
    vals = jnp.concatenate(
        [adj_values, jnp.zeros((PADN,), jnp.float32)])

    allHyper, x1, lm1 = _stage_a(embeds, zishiying, Hyper)

    s1 = _spmm_sc(x1.reshape(2 * N, HALF), cols, rows, vals)
    hyperLat1 = _stage_h(allHyper, lm1)
    x2, lm2 = _stage_cb(hyperLat1, s1.reshape(2, NP, HALF), zishiying,
                        allHyper)

    s2 = _spmm_sc(x2.reshape(2 * N, HALF), cols, rows, vals)
    hyperLat2 = _stage_h(allHyper, lm2)
    temEmbeds2 = _stage_eb(s2.reshape(2, NP, HALF))

    return (temEmbeds2, hyperLat1, hyperLat2)


# speedup vs baseline: 1.0010x; 1.0004x over previous
import jax
import jax.numpy as jnp
from jax import lax
from jax.experimental import pallas as pl
from jax.experimental.pallas import tpu as pltpu
from jax.experimental.pallas import tpu_sc as plsc

USER = 25000
ITEM = 25000
N = USER + ITEM
D = 64
H = 128
HALF = 32
LEAKY = 0.5
E = 800000

EPT = 50176
EPAD = 16 * EPT
PADN = EPAD - E
WIN = 256
NSUB = 2
NWIN = EPT // WIN
NP = 50048
RPT = NP // 16
ZR = 68

BLK = 1000
GRID = N // BLK


def _lrelu(x):
    return jnp.where(x >= 0, x, LEAKY * x)


def _spmm_body(x_hbm, cols_hbm, rows_hbm, vals_hbm, out_hbm,
               gbuf0, gbuf1, cbuf0, cbuf1, rbuf0, rbuf1, rsbuf0, rsbuf1,
               vbuf0, vbuf1, zbuf, acc, isem0, isem1, gsem0, gsem1,
               ssem0, ssem1):
    c = lax.axis_index("c")
    s = lax.axis_index("s")

    def _zrow(r, _):
        zbuf[r, pl.ds(0, 16)] = jnp.zeros((16,), jnp.float32)
        zbuf[r, pl.ds(16, 16)] = jnp.zeros((16,), jnp.float32)
        return _
    lax.fori_loop(0, ZR, _zrow, None)

    def _zcopy(j, _):
        pltpu.sync_copy(zbuf, acc.at[pl.ds(s * RPT + j * ZR, ZR)])
        return _
    lax.fori_loop(0, RPT // ZR, _zcopy, None)
    plsc.subcore_barrier()

    col_off = c * N

    def _idx_slices(w):
        base = s * EPT + w * WIN
        chunk = s * (EPT // 128) + w * NSUB
        return (vals_hbm.at[pl.ds(base, WIN)],
                cols_hbm.at[pl.ds(chunk, NSUB)],
                rows_hbm.at[pl.ds(chunk, NSUB)])

    def _issue_idx(w, cb, rb, vb, sem):
        vsrc, csrc, rsrc = _idx_slices(w)
        pltpu.async_copy(vsrc, vb, sem)
        pltpu.async_copy(csrc, cb, sem)
        pltpu.async_copy(rsrc, rb, sem)

    def _drain_idx(w, cb, rb, vb, sem):
        vsrc, csrc, rsrc = _idx_slices(w)
        pltpu.make_async_copy(vsrc, vb, sem).wait()
        pltpu.make_async_copy(csrc, cb, sem).wait()
        pltpu.make_async_copy(rsrc, rb, sem).wait()

    def _adjust(cb):
        for t in range(NSUB):
            for l in range(8):
                sl = pl.ds(l * 16, 16)
                cb[t, sl] = cb[t, sl] + col_off

    def _gathers(cb, gb, sem):
        return [
            pltpu.async_copy(x_hbm.at[cb.at[t]],
                             gb.at[pl.ds(t * 128, 128)], sem)
            for t in range(NSUB)
        ]

    def _copy_rows(rb, rsb):
        for t in range(NSUB):
            for l in range(8):
                sl = pl.ds(l * 16, 16)
                rsb[t, sl] = rb[t, sl]

    def _scale_scatter_async(gb, vb, rsb, ssem):
        for t in range(NSUB):
            @plsc.parallel_loop(t * 8, (t + 1) * 8, unroll=2)
            def _scale(g):
                v16 = vb[pl.ds(g * 16, 16)]
                for r in range(16):
                    e = g * 16 + r
                    lo = pl.ds(0, 16)
                    hi = pl.ds(16, 16)
                    gb[e, lo] = gb[e, lo] * v16[r]
                    gb[e, hi] = gb[e, hi] * v16[r]

            pltpu.async_copy(gb.at[pl.ds(t * 128, 128)],
                             acc.at[rsb.at[t]], ssem, add=True)

    def _drain_scatter(gb, rsb, ssem):
        for t in range(NSUB):
            pltpu.make_async_copy(gb.at[pl.ds(t * 128, 128)],
                                  acc.at[rsb.at[t]], ssem).wait()

    _issue_idx(0, cbuf0, rbuf0, vbuf0, isem0)

    def _pair(k, _):
        w0 = 2 * k
        w1 = w0 + 1

        @pl.when(k > 0)
        def _():
            _drain_scatter(gbuf0, rsbuf0, ssem0)
        _drain_idx(w0, cbuf0, rbuf0, vbuf0, isem0)
        _adjust(cbuf0)
        g0 = _gathers(cbuf0, gbuf0, gsem0)
        _issue_idx(w1, cbuf1, rbuf1, vbuf1, isem1)
        for d in g0:
            d.wait()
        _copy_rows(rbuf0, rsbuf0)

        @pl.when(k > 0)
        def _():
            _drain_scatter(gbuf1, rsbuf1, ssem1)
        _drain_idx(w1, cbuf1, rbuf1, vbuf1, isem1)
        _adjust(cbuf1)
        g1 = _gathers(cbuf1, gbuf1, gsem1)

        _scale_scatter_async(gbuf0, vbuf0, rsbuf0, ssem0)

        @pl.when(k < NWIN // 2 - 1)
        def _():
            _issue_idx(w0 + 2, cbuf0, rbuf0, vbuf0, isem0)

        for d in g1:
            d.wait()
        _copy_rows(rbuf1, rsbuf1)
        _scale_scatter_async(gbuf1, vbuf1, rsbuf1, ssem1)
        return _
    lax.fori_loop(0, NWIN // 2, _pair, None)
    _drain_scatter(gbuf0, rsbuf0, ssem0)
    _drain_scatter(gbuf1, rsbuf1, ssem1)
    plsc.subcore_barrier()

    pltpu.sync_copy(acc.at[pl.ds(s * RPT, RPT)],
                    out_hbm.at[pl.ds(c * NP + s * RPT, RPT)])


_spmm_sc = pl.kernel(
    _spmm_body,
    out_type=jax.ShapeDtypeStruct((2 * NP, HALF), jnp.float32),
    mesh=plsc.VectorSubcoreMesh(core_axis_name="c", subcore_axis_name="s"),
    scratch_types=[
        pltpu.VMEM((WIN, HALF), jnp.float32),
        pltpu.VMEM((WIN, HALF), jnp.float32),
        pltpu.VMEM((NSUB, 128), jnp.int32),
        pltpu.VMEM((NSUB, 128), jnp.int32),
        pltpu.VMEM((NSUB, 128), jnp.int32),
        pltpu.VMEM((NSUB, 128), jnp.int32),
        pltpu.VMEM((NSUB, 128), jnp.int32),
        pltpu.VMEM((NSUB, 128), jnp.int32),
        pltpu.VMEM((WIN,), jnp.float32),
        pltpu.VMEM((WIN,), jnp.float32),
        pltpu.VMEM((ZR, HALF), jnp.float32),
        pltpu.VMEM_SHARED((NP, HALF), jnp.float32),
        pltpu.SemaphoreType.DMA,
        pltpu.SemaphoreType.DMA,
        pltpu.SemaphoreType.DMA,
        pltpu.SemaphoreType.DMA,
        pltpu.SemaphoreType.DMA,
        pltpu.SemaphoreType.DMA,
    ],
    compiler_params=pltpu.CompilerParams(use_tc_tiling_on_sc=False),
)


def _stage_a_body(e_ref, z_ref, h_ref, ah_ref, x_ref, lm_ref):
    i = pl.program_id(0)
    e = e_ref[...]
    a = jax.nn.sigmoid(z_ref[...]) * 2.0 - 1.0
    x = e * a
    ah = jnp.dot(e, h_ref[...], preferred_element_type=jnp.float32)
    ah_ref[...] = ah
    x_ref[0] = x[:, :HALF]
    x_ref[1] = x[:, HALF:]

    @pl.when(i == 0)
    def _():
        lm_ref[...] = jnp.zeros_like(lm_ref)

    lm_ref[...] += lax.dot_general(
        ah, e, (((0,), (0,)), ((), ())), preferred_element_type=jnp.float32)

    @pl.when(i == GRID - 1)
    def _():
        lm_ref[...] = _lrelu(lm_ref[...])


_stage_a = pl.pallas_call(
    _stage_a_body,
    grid=(GRID,),
    in_specs=[
        pl.BlockSpec((BLK, D), lambda i: (i, 0)),
        pl.BlockSpec((BLK, D), lambda i: (i, 0)),
        pl.BlockSpec((D, H), lambda i: (0, 0)),
    ],
    out_specs=[
        pl.BlockSpec((BLK, H), lambda i: (i, 0)),
        pl.BlockSpec((2, BLK, HALF), lambda i: (0, i, 0)),
        pl.BlockSpec((H, D), lambda i: (0, 0)),
    ],
    out_shape=[
        jax.ShapeDtypeStruct((N, H), jnp.float32),
        jax.ShapeDtypeStruct((2, N, HALF), jnp.float32),
        jax.ShapeDtypeStruct((H, D), jnp.float32),
    ],
)


def _stage_h_body(ah_ref, lm_ref, hl_ref):
    hl_ref[...] = _lrelu(jnp.dot(ah_ref[...], lm_ref[...],
                                 preferred_element_type=jnp.float32))


_stage_h = pl.pallas_call(
    _stage_h_body,
    grid=(GRID,),
    in_specs=[
        pl.BlockSpec((BLK, H), lambda i: (i, 0)),
        pl.BlockSpec((H, D), lambda i: (0, 0)),
    ],
    out_specs=pl.BlockSpec((BLK, D), lambda i: (i, 0)),
    out_shape=jax.ShapeDtypeStruct((N, D), jnp.float32),
)


def _stage_cb_body(hl_ref, s_ref, z_ref, ah_ref, x_ref, lm_ref):
    i = pl.program_id(0)
    t = _lrelu(jnp.concatenate([s_ref[0], s_ref[1]], axis=1))
    lat = hl_ref[...] + t
    a = jax.nn.sigmoid(z_ref[...]) * 2.0 - 1.0
    x = lat * a
    x_ref[0] = x[:, :HALF]
    x_ref[1] = x[:, HALF:]

    @pl.when(i == 0)
    def _():
        lm_ref[...] = jnp.zeros_like(lm_ref)

    lm_ref[...] += lax.dot_general(
        ah_ref[...], lat, (((0,), (0,)), ((), ())),
        preferred_element_type=jnp.float32)

    @pl.when(i == GRID - 1)
    def _():
        lm_ref[...] = _lrelu(lm_ref[...])


_stage_cb = pl.pallas_call(
    _stage_cb_body,
    grid=(GRID,),
    in_specs=[
        pl.BlockSpec((BLK, D), lambda i: (i, 0)),
        pl.BlockSpec((2, BLK, HALF), lambda i: (0, i, 0)),
        pl.BlockSpec((BLK, D), lambda i: (i, 0)),
        pl.BlockSpec((BLK, H), lambda i: (i, 0)),
    ],
    out_specs=[
        pl.BlockSpec((2, BLK, HALF), lambda i: (0, i, 0)),
        pl.BlockSpec((H, D), lambda i: (0, 0)),
    ],
    out_shape=[
        jax.ShapeDtypeStruct((2, N, HALF), jnp.float32),
        jax.ShapeDtypeStruct((H, D), jnp.float32),
    ],
)


def _stage_eb_body(s_ref, te_ref):
    te_ref[...] = _lrelu(jnp.concatenate([s_ref[0], s_ref[1]], axis=1))


_stage_eb = pl.pallas_call(
    _stage_eb_body,
    grid=(GRID,),
    in_specs=[
        pl.BlockSpec((2, BLK, HALF), lambda i: (0, i, 0)),
    ],
    out_specs=pl.BlockSpec((BLK, D), lambda i: (i, 0)),
    out_shape=jax.ShapeDtypeStruct((N, D), jnp.float32),
)


def kernel(adj_indices, adj_values, keepRate, uEmbeds, iEmbeds, Hyper,
           zishiying):
    del keepRate
    embeds = jnp.concatenate([uEmbeds, iEmbeds], axis=0)

    pad_idx = (jnp.arange(PADN, dtype=jnp.int32) * 64) % N
    rows = jnp.concatenate([adj_indices[0], pad_idx]).reshape(EPAD // 128, 128)
    cols = jnp.concatenate([adj_indices[1], pad_idx]).reshape(EPAD // 128, 128)
    vals = jnp.concatenate(
        [adj_values, jnp.zeros((PADN,), jnp.float32)])

    allHyper, x1, lm1 = _stage_a(embeds, zishiying, Hyper)

    s1 = _spmm_sc(x1.reshape(2 * N, HALF), cols, rows, vals)
    hyperLat1 = _stage_h(allHyper, lm1)
    x2, lm2 = _stage_cb(hyperLat1, s1.reshape(2, NP, HALF), zishiying,
                        allHyper)

    s2 = _spmm_sc(x2.reshape(2 * N, HALF), cols, rows, vals)
    hyperLat2 = _stage_h(allHyper, lm2)
    temEmbeds2 = _stage_eb(s2.reshape(2, NP, HALF))

    return (temEmbeds2, hyperLat1, hyperLat2)


# speedup vs baseline: 1.0056x; 1.0045x over previous
import jax
import jax.numpy as jnp
from jax import lax
from jax.experimental import pallas as pl
from jax.experimental.pallas import tpu as pltpu
from jax.experimental.pallas import tpu_sc as plsc

USER = 25000
ITEM = 25000
N = USER + ITEM
D = 64
H = 128
HALF = 32
LEAKY = 0.5
E = 800000

EPT = 50176
EPAD = 16 * EPT
PADN = EPAD - E
WIN = 256
NSUB = 2
NWIN = EPT // WIN
NP = 50048
RPT = NP // 16
ZR = 68

BLK = 1000
GRID = N // BLK


def _lrelu(x):
    return jnp.where(x >= 0, x, LEAKY * x)


def _spmm_body(x_hbm, cols_hbm, rows_hbm, vals_hbm, out_hbm,
               gbuf0, gbuf1, cbuf0, cbuf1, rbuf0, rbuf1, rsbuf0, rsbuf1,
               vbuf0, vbuf1, zbuf, acc, isem0, isem1, gsem0, gsem1,
               ssem0, ssem1):
    c = lax.axis_index("c")
    s = lax.axis_index("s")

    def _zrow(r, _):
        zbuf[r, pl.ds(0, 16)] = jnp.zeros((16,), jnp.float32)
        zbuf[r, pl.ds(16, 16)] = jnp.zeros((16,), jnp.float32)
        return _
    lax.fori_loop(0, ZR, _zrow, None)

    def _zcopy(j, _):
        pltpu.sync_copy(zbuf, acc.at[pl.ds(s * RPT + j * ZR, ZR)])
        return _
    lax.fori_loop(0, RPT // ZR, _zcopy, None)
    plsc.subcore_barrier()

    col_off = c * N

    def _idx_slices(w):
        base = s * EPT + w * WIN
        chunk = s * (EPT // 128) + w * NSUB
        return (vals_hbm.at[pl.ds(base, WIN)],
                cols_hbm.at[pl.ds(chunk, NSUB)],
                rows_hbm.at[pl.ds(chunk, NSUB)])

    def _issue_idx(w, cb, rb, vb, sem):
        vsrc, csrc, rsrc = _idx_slices(w)
        pltpu.async_copy(vsrc, vb, sem)
        pltpu.async_copy(csrc, cb, sem)
        pltpu.async_copy(rsrc, rb, sem)

    def _drain_idx(w, cb, rb, vb, sem):
        vsrc, csrc, rsrc = _idx_slices(w)
        pltpu.make_async_copy(vsrc, vb, sem).wait()
        pltpu.make_async_copy(csrc, cb, sem).wait()
        pltpu.make_async_copy(rsrc, rb, sem).wait()

    def _adjust(cb):
        for t in range(NSUB):
            for l in range(8):
                sl = pl.ds(l * 16, 16)
                cb[t, sl] = cb[t, sl] + col_off

    def _gathers(cb, gb, sem):
        return [
            pltpu.async_copy(x_hbm.at[cb.at[t]],
                             gb.at[pl.ds(t * 128, 128)], sem)
            for t in range(NSUB)
        ]

    def _copy_rows(rb, rsb):
        for t in range(NSUB):
            for l in range(8):
                sl = pl.ds(l * 16, 16)
                rsb[t, sl] = rb[t, sl]

    def _scale_scatter_async(gb, vb, rsb, ssem):
        for t in range(NSUB):
            @plsc.parallel_loop(t * 8, (t + 1) * 8, unroll=2)
            def _scale(g):
                v16 = vb[pl.ds(g * 16, 16)]
                for r in range(16):
                    e = g * 16 + r
                    lo = pl.ds(0, 16)
                    hi = pl.ds(16, 16)
                    gb[e, lo] = gb[e, lo] * v16[r]
                    gb[e, hi] = gb[e, hi] * v16[r]

            pltpu.async_copy(gb.at[pl.ds(t * 128, 128)],
                             acc.at[rsb.at[t]], ssem, add=True)

    def _drain_scatter(gb, rsb, ssem):
        for t in range(NSUB):
            pltpu.make_async_copy(gb.at[pl.ds(t * 128, 128)],
                                  acc.at[rsb.at[t]], ssem).wait()

    _issue_idx(0, cbuf0, rbuf0, vbuf0, isem0)

    def _pair(k, _):
        w0 = 2 * k
        w1 = w0 + 1

        @pl.when(k > 0)
        def _():
            _drain_scatter(gbuf0, rsbuf0, ssem0)
        _drain_idx(w0, cbuf0, rbuf0, vbuf0, isem0)
        _adjust(cbuf0)
        g0 = _gathers(cbuf0, gbuf0, gsem0)
        _issue_idx(w1, cbuf1, rbuf1, vbuf1, isem1)
        for d in g0:
            d.wait()
        _copy_rows(rbuf0, rsbuf0)

        @pl.when(k > 0)
        def _():
            _drain_scatter(gbuf1, rsbuf1, ssem1)
        _drain_idx(w1, cbuf1, rbuf1, vbuf1, isem1)
        _adjust(cbuf1)
        g1 = _gathers(cbuf1, gbuf1, gsem1)

        _scale_scatter_async(gbuf0, vbuf0, rsbuf0, ssem0)

        @pl.when(k < NWIN // 2 - 1)
        def _():
            _issue_idx(w0 + 2, cbuf0, rbuf0, vbuf0, isem0)

        for d in g1:
            d.wait()
        _copy_rows(rbuf1, rsbuf1)
        _scale_scatter_async(gbuf1, vbuf1, rsbuf1, ssem1)
        return _
    lax.fori_loop(0, NWIN // 2, _pair, None)
    _drain_scatter(gbuf0, rsbuf0, ssem0)
    _drain_scatter(gbuf1, rsbuf1, ssem1)
    plsc.subcore_barrier()

    pltpu.sync_copy(acc.at[pl.ds(s * RPT, RPT)],
                    out_hbm.at[pl.ds(c * NP + s * RPT, RPT)])


_spmm_sc = pl.kernel(
    _spmm_body,
    out_type=jax.ShapeDtypeStruct((2 * NP, HALF), jnp.float32),
    mesh=plsc.VectorSubcoreMesh(core_axis_name="c", subcore_axis_name="s"),
    scratch_types=[
        pltpu.VMEM((WIN, HALF), jnp.float32),
        pltpu.VMEM((WIN, HALF), jnp.float32),
        pltpu.VMEM((NSUB, 128), jnp.int32),
        pltpu.VMEM((NSUB, 128), jnp.int32),
        pltpu.VMEM((NSUB, 128), jnp.int32),
        pltpu.VMEM((NSUB, 128), jnp.int32),
        pltpu.VMEM((NSUB, 128), jnp.int32),
        pltpu.VMEM((NSUB, 128), jnp.int32),
        pltpu.VMEM((WIN,), jnp.float32),
        pltpu.VMEM((WIN,), jnp.float32),
        pltpu.VMEM((ZR, HALF), jnp.float32),
        pltpu.VMEM_SHARED((NP, HALF), jnp.float32),
        pltpu.SemaphoreType.DMA,
        pltpu.SemaphoreType.DMA,
        pltpu.SemaphoreType.DMA,
        pltpu.SemaphoreType.DMA,
        pltpu.SemaphoreType.DMA,
        pltpu.SemaphoreType.DMA,
    ],
    compiler_params=pltpu.CompilerParams(use_tc_tiling_on_sc=False),
)


def _stage_a_body(e_ref, z_ref, h_ref, ah_ref, x_ref):
    e = e_ref[...]
    a = jax.nn.sigmoid(z_ref[...]) * 2.0 - 1.0
    x = e * a
    ah_ref[...] = jnp.dot(e, h_ref[...], preferred_element_type=jnp.float32)
    x_ref[0] = x[:, :HALF]
    x_ref[1] = x[:, HALF:]


_stage_a = pl.pallas_call(
    _stage_a_body,
    grid=(GRID,),
    in_specs=[
        pl.BlockSpec((BLK, D), lambda i: (i, 0)),
        pl.BlockSpec((BLK, D), lambda i: (i, 0)),
        pl.BlockSpec((D, H), lambda i: (0, 0)),
    ],
    out_specs=[
        pl.BlockSpec((BLK, H), lambda i: (i, 0)),
        pl.BlockSpec((2, BLK, HALF), lambda i: (0, i, 0)),
    ],
    out_shape=[
        jax.ShapeDtypeStruct((N, H), jnp.float32),
        jax.ShapeDtypeStruct((2, N, HALF), jnp.float32),
    ],
)


def _reduce_body(ah_ref, lat_ref, out_ref):
    i = pl.program_id(0)

    @pl.when(i == 0)
    def _():
        out_ref[...] = jnp.zeros_like(out_ref)

    out_ref[...] += lax.dot_general(
        ah_ref[...], lat_ref[...], (((0,), (0,)), ((), ())),
        preferred_element_type=jnp.float32)

    @pl.when(i == GRID - 1)
    def _():
        out_ref[...] = _lrelu(out_ref[...])


_lat_mid = pl.pallas_call(
    _reduce_body,
    grid=(GRID,),
    in_specs=[
        pl.BlockSpec((BLK, H), lambda i: (i, 0)),
        pl.BlockSpec((BLK, D), lambda i: (i, 0)),
    ],
    out_specs=pl.BlockSpec((H, D), lambda i: (0, 0)),
    out_shape=jax.ShapeDtypeStruct((H, D), jnp.float32),
)


def _stage_h_body(ah_ref, lm_ref, hl_ref):
    hl_ref[...] = _lrelu(jnp.dot(ah_ref[...], lm_ref[...],
                                 preferred_element_type=jnp.float32))


_stage_h = pl.pallas_call(
    _stage_h_body,
    grid=(GRID,),
    in_specs=[
        pl.BlockSpec((BLK, H), lambda i: (i, 0)),
        pl.BlockSpec((H, D), lambda i: (0, 0)),
    ],
    out_specs=pl.BlockSpec((BLK, D), lambda i: (i, 0)),
    out_shape=jax.ShapeDtypeStruct((N, D), jnp.float32),
)


def _stage_cb_body(hl_ref, s_ref, z_ref, lat_ref, x_ref):
    t = _lrelu(jnp.concatenate([s_ref[0], s_ref[1]], axis=1))
    lat = hl_ref[...] + t
    a = jax.nn.sigmoid(z_ref[...]) * 2.0 - 1.0
    x = lat * a
    lat_ref[...] = lat
    x_ref[0] = x[:, :HALF]
    x_ref[1] = x[:, HALF:]


_stage_cb = pl.pallas_call(
    _stage_cb_body,
    grid=(GRID,),
    in_specs=[
        pl.BlockSpec((BLK, D), lambda i: (i, 0)),
        pl.BlockSpec((2, BLK, HALF), lambda i: (0, i, 0)),
        pl.BlockSpec((BLK, D), lambda i: (i, 0)),
    ],
    out_specs=[
        pl.BlockSpec((BLK, D), lambda i: (i, 0)),
        pl.BlockSpec((2, BLK, HALF), lambda i: (0, i, 0)),
    ],
    out_shape=[
        jax.ShapeDtypeStruct((N, D), jnp.float32),
        jax.ShapeDtypeStruct((2, N, HALF), jnp.float32),
    ],
)


def _stage_eb_body(s_ref, te_ref):
    te_ref[...] = _lrelu(jnp.concatenate([s_ref[0], s_ref[1]], axis=1))


_stage_eb = pl.pallas_call(
    _stage_eb_body,
    grid=(GRID,),
    in_specs=[
        pl.BlockSpec((2, BLK, HALF), lambda i: (0, i, 0)),
    ],
    out_specs=pl.BlockSpec((BLK, D), lambda i: (i, 0)),
    out_shape=jax.ShapeDtypeStruct((N, D), jnp.float32),
)


def kernel(adj_indices, adj_values, keepRate, uEmbeds, iEmbeds, Hyper,
           zishiying):
    del keepRate
    embeds = jnp.concatenate([uEmbeds, iEmbeds], axis=0)

    pad_idx = (jnp.arange(PADN, dtype=jnp.int32) * 64) % N
    rows = jnp.concatenate([adj_indices[0], pad_idx]).reshape(EPAD // 128, 128)
    cols = jnp.concatenate([adj_indices[1], pad_idx]).reshape(EPAD // 128, 128)
    vals = jnp.concatenate(
        [adj_values, jnp.zeros((PADN,), jnp.float32)])

    allHyper, x1 = _stage_a(embeds, zishiying, Hyper)

    s1 = _spmm_sc(x1.reshape(2 * N, HALF), cols, rows, vals)
    lm1 = _lat_mid(allHyper, embeds)
    hyperLat1 = _stage_h(allHyper, lm1)
    lat1, x2 = _stage_cb(hyperLat1, s1.reshape(2, NP, HALF), zishiying)

    s2 = _spmm_sc(x2.reshape(2 * N, HALF), cols, rows, vals)
    lm2 = _lat_mid(allHyper, lat1)
    hyperLat2 = _stage_h(allHyper, lm2)
    temEmbeds2 = _stage_eb(s2.reshape(2, NP, HALF))

    return (temEmbeds2, hyperLat1, hyperLat2)
